# Initial kernel scaffold; baseline (speedup 1.0000x reference)
import jax
import jax.numpy as jnp
from jax.experimental import pallas as pl


def kernel(x, edge_index, weight_local, w1, b1, w2, b2, alpha, fc_w, fc_b):
    raise NotImplementedError("write your pallas kernel here")


# speedup vs baseline: 10.0700x; 10.0700x over previous
import functools

import jax
import jax.numpy as jnp
from jax import lax
from jax.experimental import pallas as pl
from jax.experimental.pallas import tpu as pltpu
from jax.experimental.pallas import tpu_sc as plsc

NC = 2
NS = 16
NW = NC * NS
K = 128
TRASH = 16


def _make_hist(n, nch):
    zr = (n // NS) & ~7
    tail = n - zr * NS
    mesh = plsc.VectorSubcoreMesh(core_axis_name="c", subcore_axis_name="s")

    @functools.partial(
        pl.kernel, mesh=mesh,
        out_type=jax.ShapeDtypeStruct((NC * n,), jnp.float32),
        scratch_types=[
            pltpu.VMEM_SHARED((n + TRASH,), jnp.float32),
            pltpu.VMEM((nch, K), jnp.int32),
            pltpu.VMEM((K,), jnp.float32),
            pltpu.VMEM((zr,), jnp.float32),
        ],
    )
    def hist_k(col_hbm, out_hbm, hist_sh, col_v, ones_v, z_v):
        core = lax.axis_index("c")
        sub = lax.axis_index("s")
        wid = sub * NC + core

        def fill_ones(i, _):
            ones_v[pl.ds(i * 16, 16)] = jnp.ones((16,), jnp.float32)
            return 0
        lax.fori_loop(0, K // 16, fill_ones, 0)

        def fill_zero(i, _):
            z_v[pl.ds(i * 16, 16)] = jnp.zeros((16,), jnp.float32)
            return 0
        lax.fori_loop(0, zr // 16, fill_zero, 0)

        pltpu.sync_copy(z_v, hist_sh.at[pl.ds(sub * zr, zr)])
        if tail:
            @pl.when(sub == 0)
            def _():
                pltpu.sync_copy(z_v.at[pl.ds(0, tail)],
                                hist_sh.at[pl.ds(zr * NS, tail)])
        pltpu.sync_copy(col_hbm.at[wid], col_v)
        plsc.subcore_barrier()

        def body(g, _):
            pltpu.sync_copy(ones_v, hist_sh.at[col_v.at[g]], add=True)
            return 0
        lax.fori_loop(0, nch, body, 0)
        plsc.subcore_barrier()

        pltpu.sync_copy(hist_sh.at[pl.ds(sub * zr, zr)], z_v)
        pltpu.sync_copy(z_v, out_hbm.at[pl.ds(core * n + sub * zr, zr)])
        if tail:
            @pl.when(sub == 0)
            def _():
                pltpu.sync_copy(hist_sh.at[pl.ds(zr * NS, tail)],
                                z_v.at[pl.ds(0, tail)])
                pltpu.sync_copy(z_v.at[pl.ds(0, tail)],
                                out_hbm.at[pl.ds(core * n + zr * NS, tail)])

    return hist_k


def _make_scatter(n, nch, d):
    assert nch % 4 == 0
    hch = nch // 2
    zr = 32
    cr = (n // NS) & ~7
    nz = cr // zr
    zrem = cr - nz * zr
    tail = n - cr * NS
    mesh = plsc.VectorSubcoreMesh(core_axis_name="c", subcore_axis_name="s")

    @functools.partial(
        pl.kernel, mesh=mesh,
        out_type=jax.ShapeDtypeStruct((NC, n, d), jnp.float32),
        scratch_types=[
            pltpu.VMEM_SHARED((n + TRASH, d), jnp.float32),
            pltpu.VMEM((hch, K), jnp.int32),
            pltpu.VMEM((hch, K), jnp.int32),
            pltpu.VMEM((2, K, d), jnp.float32),
            pltpu.VMEM((zr, d), jnp.float32),
            pltpu.SemaphoreType.DMA,
            pltpu.SemaphoreType.DMA,
        ],
    )
    def scat_k(xs_hbm, row_hbm, col_hbm, out_hbm,
               acc_sh, row_v, col_v, bufs, z_v, sem0, sem1):
        core = lax.axis_index("c")
        sub = lax.axis_index("s")
        wid = sub * NC + core
        sems = (sem0, sem1)

        def zfill(r, _):
            for l in range(d // 16):
                z_v[r, pl.ds(l * 16, 16)] = jnp.zeros((16,), jnp.float32)
            return 0
        lax.fori_loop(0, zr, zfill, 0)

        def zcopy(j, _):
            pltpu.sync_copy(z_v, acc_sh.at[pl.ds(sub * cr + j * zr, zr)])
            return 0
        lax.fori_loop(0, nz, zcopy, 0)
        if zrem:
            pltpu.sync_copy(z_v.at[pl.ds(0, zrem)],
                            acc_sh.at[pl.ds(sub * cr + nz * zr, zrem)])
        if tail:
            @pl.when(sub == 0)
            def _():
                pltpu.sync_copy(z_v.at[pl.ds(0, tail)],
                                acc_sh.at[pl.ds(cr * NS, tail)])
        plsc.subcore_barrier()

        for h in range(2):
            pltpu.sync_copy(row_hbm.at[wid, pl.ds(h * hch, hch)], row_v)
            pltpu.sync_copy(col_hbm.at[wid, pl.ds(h * hch, hch)], col_v)

            pltpu.async_copy(xs_hbm.at[row_v.at[0]], bufs.at[0], sem0)
            pltpu.async_copy(xs_hbm.at[row_v.at[1]], bufs.at[1], sem1)

            def body(o, _):
                for b in range(2):
                    g = o * 2 + b
                    pltpu.make_async_copy(
                        xs_hbm.at[row_v.at[g]], bufs.at[b], sems[b]).wait()
                    pltpu.sync_copy(bufs.at[b], acc_sh.at[col_v.at[g]],
                                    add=True)

                    @pl.when(g + 2 < hch)
                    def _():
                        pltpu.async_copy(
                            xs_hbm.at[row_v.at[g + 2]], bufs.at[b], sems[b])
                return 0
            lax.fori_loop(0, hch // 2, body, 0)
        plsc.subcore_barrier()

        pltpu.sync_copy(acc_sh.at[pl.ds(sub * cr, cr)],
                        out_hbm.at[core, pl.ds(sub * cr, cr)])
        if tail:
            @pl.when(sub == 0)
            def _():
                pltpu.sync_copy(acc_sh.at[pl.ds(cr * NS, tail)],
                                out_hbm.at[core, pl.ds(cr * NS, tail)])

    return scat_k


def _make_prep(n, d, bn):
    nb = n // bn

    def prep_k(x_ref, hist_ref, wl_ref, fcw_ref, w1_ref, b1_ref, w2_ref,
               b2_ref, alpha_ref, fcb_ref, xs_ref, s_ref, wcs_ref, rv_ref,
               acc):
        i = pl.program_id(0)
        deg = jnp.sum(hist_ref[0], axis=1, keepdims=True)
        s = jnp.where(deg > 0.0, lax.rsqrt(jnp.maximum(deg, 1e-30)), 0.0)
        xb = x_ref[...]
        xs_ref[...] = xb * s
        s_ref[...] = s

        psum = jnp.sum(xb, axis=0, keepdims=True)

        @pl.when(i == 0)
        def _():
            acc[0:1, :] = psum

        @pl.when(i > 0)
        def _():
            acc[0:1, :] = acc[0:1, :] + psum

        @pl.when(i == nb - 1)
        def _():
            mean = acc[0:1, :] * (1.0 / n)
            sig = jax.nn.sigmoid(alpha_ref[...])
            cdims = (((1,), (1,)), ((), ()))
            h = jnp.maximum(
                lax.dot_general(mean, w1_ref[...], cdims,
                                preferred_element_type=jnp.float32)
                + b1_ref[...], 0.0)
            g = lax.dot_general(h, w2_ref[...], cdims,
                                preferred_element_type=jnp.float32) + b2_ref[...]
            gf = lax.dot_general(g, fcw_ref[...], cdims,
                                 preferred_element_type=jnp.float32)
            rv_ref[...] = (1.0 - sig) * gf + fcb_ref[...]
            wcs_ref[...] = sig * lax.dot_general(
                wl_ref[...], fcw_ref[...], cdims,
                preferred_element_type=jnp.float32)

    full = lambda i: (0, 0)
    return pl.pallas_call(
        prep_k,
        grid=(nb,),
        in_specs=[
            pl.BlockSpec((bn, d), lambda i: (i, 0)),
            pl.BlockSpec((1, bn, 2), lambda i: (i, 0, 0)),
            pl.BlockSpec((d, d), full),
            pl.BlockSpec((d, d), full),
            pl.BlockSpec((d, d), full),
            pl.BlockSpec((1, d), full),
            pl.BlockSpec((d, d), full),
            pl.BlockSpec((1, d), full),
            pl.BlockSpec((1, 1), full),
            pl.BlockSpec((1, d), full),
        ],
        out_specs=[
            pl.BlockSpec((bn, d), lambda i: (i, 0)),
            pl.BlockSpec((bn, 1), lambda i: (i, 0)),
            pl.BlockSpec((d, d), full),
            pl.BlockSpec((1, d), full),
        ],
        out_shape=[
            jax.ShapeDtypeStruct((n, d), jnp.float32),
            jax.ShapeDtypeStruct((n, 1), jnp.float32),
            jax.ShapeDtypeStruct((d, d), jnp.float32),
            jax.ShapeDtypeStruct((1, d), jnp.float32),
        ],
        scratch_shapes=[pltpu.VMEM((8, d), jnp.float32)],
    )


def _make_final(n, d, bn):
    nb = n // bn

    def fin_k(acc_ref, s_ref, wcs_ref, rv_ref, out_ref):
        a = (acc_ref[0] + acc_ref[1]) * s_ref[...]
        out_ref[...] = jnp.dot(a, wcs_ref[...],
                               preferred_element_type=jnp.float32) + rv_ref[...]

    full = lambda i: (0, 0)
    return pl.pallas_call(
        fin_k,
        grid=(nb,),
        in_specs=[
            pl.BlockSpec((NC, bn, d), lambda i: (0, i, 0)),
            pl.BlockSpec((bn, 1), lambda i: (i, 0)),
            pl.BlockSpec((d, d), full),
            pl.BlockSpec((1, d), full),
        ],
        out_specs=pl.BlockSpec((bn, d), lambda i: (i, 0)),
        out_shape=jax.ShapeDtypeStruct((n, d), jnp.float32),
    )


def kernel(x, edge_index, weight_local, w1, b1, w2, b2, alpha, fc_w, fc_b):
    n, d = x.shape
    e = edge_index.shape[1]
    bn = 1000
    nb = n // bn

    row = edge_index[0].astype(jnp.int32)
    col = edge_index[1].astype(jnp.int32)

    nch = -(-e // (NW * K))
    nch += (-nch) % 4
    e_pad = NW * K * nch
    pad = e_pad - e
    if pad:
        row = jnp.concatenate([row, jnp.zeros((pad,), jnp.int32)])
        col = jnp.concatenate(
            [col, n + (jnp.arange(pad, dtype=jnp.int32) % TRASH)])
    row3 = row.reshape(NW, nch, K)
    col3 = col.reshape(NW, nch, K)

    hists = _make_hist(n, nch)(col3).reshape(NC, n)
    histt = jnp.transpose(hists).reshape(nb, bn, 2)

    xs, s, wcs, rv = _make_prep(n, d, bn)(
        x, histt, weight_local, fc_w, w1, b1.reshape(1, d), w2,
        b2.reshape(1, d), alpha.reshape(1, 1), fc_b.reshape(1, d))

    accp = _make_scatter(n, nch, d)(xs, row3, col3)

    return _make_final(n, d, bn)(accp, s, wcs, rv)


# speedup vs baseline: 26.8740x; 2.6687x over previous
import functools

import jax
import jax.numpy as jnp
from jax import lax
from jax.experimental import pallas as pl
from jax.experimental.pallas import tpu as pltpu
from jax.experimental.pallas import tpu_sc as plsc

NC = 2
NS = 16
NW = NC * NS
K = 128
TRASH = 16


def _make_hist(n, nch):
    zr = (n // NS) & ~7
    tail = n - zr * NS
    mesh = plsc.VectorSubcoreMesh(core_axis_name="c", subcore_axis_name="s")

    @functools.partial(
        pl.kernel, mesh=mesh,
        out_type=jax.ShapeDtypeStruct((NC * n,), jnp.float32),
        scratch_types=[
            pltpu.VMEM_SHARED((n + TRASH,), jnp.float32),
            pltpu.VMEM((nch, K), jnp.int32),
            pltpu.VMEM((K,), jnp.float32),
            pltpu.VMEM((zr,), jnp.float32),
        ],
    )
    def hist_k(col_hbm, out_hbm, hist_sh, col_v, ones_v, z_v):
        core = lax.axis_index("c")
        sub = lax.axis_index("s")
        wid = sub * NC + core

        def fill_ones(i, _):
            ones_v[pl.ds(i * 16, 16)] = jnp.ones((16,), jnp.float32)
            return 0
        lax.fori_loop(0, K // 16, fill_ones, 0)

        def fill_zero(i, _):
            z_v[pl.ds(i * 16, 16)] = jnp.zeros((16,), jnp.float32)
            return 0
        lax.fori_loop(0, zr // 16, fill_zero, 0)

        pltpu.sync_copy(z_v, hist_sh.at[pl.ds(sub * zr, zr)])
        if tail:
            @pl.when(sub == 0)
            def _():
                pltpu.sync_copy(z_v.at[pl.ds(0, tail)],
                                hist_sh.at[pl.ds(zr * NS, tail)])
        pltpu.sync_copy(col_hbm.at[wid], col_v)
        plsc.subcore_barrier()

        def body(g, _):
            pltpu.sync_copy(ones_v, hist_sh.at[col_v.at[g]], add=True)
            return 0
        lax.fori_loop(0, nch, body, 0)
        plsc.subcore_barrier()

        pltpu.sync_copy(hist_sh.at[pl.ds(sub * zr, zr)], z_v)
        pltpu.sync_copy(z_v, out_hbm.at[pl.ds(core * n + sub * zr, zr)])
        if tail:
            @pl.when(sub == 0)
            def _():
                pltpu.sync_copy(hist_sh.at[pl.ds(zr * NS, tail)],
                                z_v.at[pl.ds(0, tail)])
                pltpu.sync_copy(z_v.at[pl.ds(0, tail)],
                                out_hbm.at[pl.ds(core * n + zr * NS, tail)])

    return hist_k


def _make_scatter(n, nch, d):
    assert nch % 4 == 0
    hch = nch // 2
    zr = 32
    cr = (n // NS) & ~7
    nz = cr // zr
    zrem = cr - nz * zr
    tail = n - cr * NS
    mesh = plsc.VectorSubcoreMesh(core_axis_name="c", subcore_axis_name="s")

    @functools.partial(
        pl.kernel, mesh=mesh,
        out_type=jax.ShapeDtypeStruct((NC, n, d), jnp.float32),
        scratch_types=[
            pltpu.VMEM_SHARED((n, d), jnp.float32),
            pltpu.VMEM((hch, K), jnp.int32),
            pltpu.VMEM((hch, K), jnp.int32),
            pltpu.VMEM((2, K, d), jnp.float32),
            pltpu.VMEM((zr, d), jnp.float32),
            pltpu.SemaphoreType.DMA,
            pltpu.SemaphoreType.DMA,
        ],
    )
    def scat_k(xs_hbm, row_hbm, col_hbm, out_hbm,
               acc_sh, row_v, col_v, bufs, z_v, sem0, sem1):
        core = lax.axis_index("c")
        sub = lax.axis_index("s")
        wid = sub * NC + core
        sems = (sem0, sem1)

        def zfill(r, _):
            for l in range(d // 16):
                z_v[r, pl.ds(l * 16, 16)] = jnp.zeros((16,), jnp.float32)
            return 0
        lax.fori_loop(0, zr, zfill, 0)

        def zcopy(j, _):
            pltpu.sync_copy(z_v, acc_sh.at[pl.ds(sub * cr + j * zr, zr)])
            return 0
        lax.fori_loop(0, nz, zcopy, 0)
        if zrem:
            pltpu.sync_copy(z_v.at[pl.ds(0, zrem)],
                            acc_sh.at[pl.ds(sub * cr + nz * zr, zrem)])
        if tail:
            @pl.when(sub == 0)
            def _():
                pltpu.sync_copy(z_v.at[pl.ds(0, tail)],
                                acc_sh.at[pl.ds(cr * NS, tail)])
        plsc.subcore_barrier()

        for h in range(2):
            pltpu.sync_copy(row_hbm.at[wid, pl.ds(h * hch, hch)], row_v)
            pltpu.sync_copy(col_hbm.at[wid, pl.ds(h * hch, hch)], col_v)

            pltpu.async_copy(xs_hbm.at[row_v.at[0]], bufs.at[0], sem0)
            pltpu.async_copy(xs_hbm.at[row_v.at[1]], bufs.at[1], sem1)

            def body(o, _):
                for b in range(2):
                    g = o * 2 + b
                    pltpu.make_async_copy(
                        xs_hbm.at[row_v.at[g]], bufs.at[b], sems[b]).wait()
                    pltpu.sync_copy(bufs.at[b], acc_sh.at[col_v.at[g]],
                                    add=True)

                    @pl.when(g + 2 < hch)
                    def _():
                        pltpu.async_copy(
                            xs_hbm.at[row_v.at[g + 2]], bufs.at[b], sems[b])
                return 0
            lax.fori_loop(0, hch // 2, body, 0)
        plsc.subcore_barrier()

        pltpu.sync_copy(acc_sh.at[pl.ds(sub * cr, cr)],
                        out_hbm.at[core, pl.ds(sub * cr, cr)])
        if tail:
            @pl.when(sub == 0)
            def _():
                pltpu.sync_copy(acc_sh.at[pl.ds(cr * NS, tail)],
                                out_hbm.at[core, pl.ds(cr * NS, tail)])

    return scat_k


def _make_prep(n, d, bn):
    nb = n // bn

    def prep_k(x_ref, hist_ref, wl_ref, fcw_ref, w1_ref, b1_ref, w2_ref,
               b2_ref, alpha_ref, fcb_ref, xs_ref, s_ref, wcs_ref, rv_ref,
               acc):
        i = pl.program_id(0)
        deg = jnp.sum(hist_ref[0], axis=1, keepdims=True)
        s = jnp.where(deg > 0.0, lax.rsqrt(jnp.maximum(deg, 1e-30)), 0.0)
        xb = x_ref[...]
        xs_ref[...] = xb * s
        s_ref[...] = s

        psum = jnp.sum(xb, axis=0, keepdims=True)

        @pl.when(i == 0)
        def _():
            acc[0:1, :] = psum

        @pl.when(i > 0)
        def _():
            acc[0:1, :] = acc[0:1, :] + psum

        @pl.when(i == nb - 1)
        def _():
            mean = acc[0:1, :] * (1.0 / n)
            sig = jax.nn.sigmoid(alpha_ref[...])
            cdims = (((1,), (1,)), ((), ()))
            h = jnp.maximum(
                lax.dot_general(mean, w1_ref[...], cdims,
                                preferred_element_type=jnp.float32)
                + b1_ref[...], 0.0)
            g = lax.dot_general(h, w2_ref[...], cdims,
                                preferred_element_type=jnp.float32) + b2_ref[...]
            gf = lax.dot_general(g, fcw_ref[...], cdims,
                                 preferred_element_type=jnp.float32)
            rv_ref[...] = (1.0 - sig) * gf + fcb_ref[...]
            wcs_ref[...] = sig * lax.dot_general(
                wl_ref[...], fcw_ref[...], cdims,
                preferred_element_type=jnp.float32)

    full = lambda i: (0, 0)
    return pl.pallas_call(
        prep_k,
        grid=(nb,),
        in_specs=[
            pl.BlockSpec((bn, d), lambda i: (i, 0)),
            pl.BlockSpec((1, bn, 2), lambda i: (i, 0, 0)),
            pl.BlockSpec((d, d), full),
            pl.BlockSpec((d, d), full),
            pl.BlockSpec((d, d), full),
            pl.BlockSpec((1, d), full),
            pl.BlockSpec((d, d), full),
            pl.BlockSpec((1, d), full),
            pl.BlockSpec((1, 1), full),
            pl.BlockSpec((1, d), full),
        ],
        out_specs=[
            pl.BlockSpec((bn, d), lambda i: (i, 0)),
            pl.BlockSpec((bn, 1), lambda i: (i, 0)),
            pl.BlockSpec((d, d), full),
            pl.BlockSpec((1, d), full),
        ],
        out_shape=[
            jax.ShapeDtypeStruct((n, d), jnp.float32),
            jax.ShapeDtypeStruct((n, 1), jnp.float32),
            jax.ShapeDtypeStruct((d, d), jnp.float32),
            jax.ShapeDtypeStruct((1, d), jnp.float32),
        ],
        scratch_shapes=[pltpu.VMEM((8, d), jnp.float32)],
    )


def _make_final(n, d, bn):
    nb = n // bn

    def fin_k(acc_ref, s_ref, wcs_ref, rv_ref, out_ref):
        a = (acc_ref[0] + acc_ref[1]) * s_ref[...]
        out_ref[...] = jnp.dot(a, wcs_ref[...],
                               preferred_element_type=jnp.float32) + rv_ref[...]

    full = lambda i: (0, 0)
    return pl.pallas_call(
        fin_k,
        grid=(nb,),
        in_specs=[
            pl.BlockSpec((NC, bn, d), lambda i: (0, i, 0)),
            pl.BlockSpec((bn, 1), lambda i: (i, 0)),
            pl.BlockSpec((d, d), full),
            pl.BlockSpec((1, d), full),
        ],
        out_specs=pl.BlockSpec((bn, d), lambda i: (i, 0)),
        out_shape=jax.ShapeDtypeStruct((n, d), jnp.float32),
    )


def kernel(x, edge_index, weight_local, w1, b1, w2, b2, alpha, fc_w, fc_b):
    n, d = x.shape
    e = edge_index.shape[1]
    bn = 1000
    nb = n // bn

    row = edge_index[0].astype(jnp.int32)
    col = edge_index[1].astype(jnp.int32)

    nch = -(-e // (NW * K))
    nch += (-nch) % 4
    e_pad = NW * K * nch
    pad = e_pad - e
    if pad:
        ar = jnp.arange(pad, dtype=jnp.int32)
        row_p = jnp.concatenate([row, n + (ar % 8)])
        col_h = jnp.concatenate([col, n + (ar % TRASH)])
        col_s = jnp.concatenate([col, ar % n])
    else:
        row_p, col_h, col_s = row, col, col
    row3 = row_p.reshape(NW, nch, K)
    col_h3 = col_h.reshape(NW, nch, K)
    col_s3 = col_s.reshape(NW, nch, K)

    hists = _make_hist(n, nch)(col_h3).reshape(NC, n)
    histt = jnp.transpose(hists).reshape(nb, bn, 2)

    xs, s, wcs, rv = _make_prep(n, d, bn)(
        x, histt, weight_local, fc_w, w1, b1.reshape(1, d), w2,
        b2.reshape(1, d), alpha.reshape(1, 1), fc_b.reshape(1, d))

    xs_z = jnp.concatenate([xs, jnp.zeros((8, d), jnp.float32)], axis=0)

    accp = _make_scatter(n, nch, d)(xs_z, row3, col_s3)

    return _make_final(n, d, bn)(accp, s, wcs, rv)


# speedup vs baseline: 27.9129x; 1.0387x over previous
import functools

import jax
import jax.numpy as jnp
from jax import lax
from jax.experimental import pallas as pl
from jax.experimental.pallas import tpu as pltpu
from jax.experimental.pallas import tpu_sc as plsc

NC = 2
NS = 16
NW = NC * NS
K = 128
TRASH = 16


def _make_hist(n, nch):
    zr = (n // NS) & ~7
    tail = n - zr * NS
    mesh = plsc.VectorSubcoreMesh(core_axis_name="c", subcore_axis_name="s")

    @functools.partial(
        pl.kernel, mesh=mesh,
        out_type=jax.ShapeDtypeStruct((NC * n,), jnp.float32),
        scratch_types=[
            pltpu.VMEM_SHARED((n,), jnp.float32),
            pltpu.VMEM((nch, K), jnp.int32),
            pltpu.VMEM((K,), jnp.float32),
            pltpu.VMEM((zr,), jnp.float32),
        ],
    )
    def hist_k(col_hbm, out_hbm, hist_sh, col_v, ones_v, z_v):
        core = lax.axis_index("c")
        sub = lax.axis_index("s")
        wid = sub * NC + core

        def fill_ones(i, _):
            ones_v[pl.ds(i * 16, 16)] = jnp.ones((16,), jnp.float32)
            return 0
        lax.fori_loop(0, K // 16, fill_ones, 0)

        def fill_zero(i, _):
            z_v[pl.ds(i * 16, 16)] = jnp.zeros((16,), jnp.float32)
            return 0
        lax.fori_loop(0, zr // 16, fill_zero, 0)

        pltpu.sync_copy(z_v, hist_sh.at[pl.ds(sub * zr, zr)])
        if tail:
            @pl.when(sub == 0)
            def _():
                pltpu.sync_copy(z_v.at[pl.ds(0, tail)],
                                hist_sh.at[pl.ds(zr * NS, tail)])
        pltpu.sync_copy(col_hbm.at[wid], col_v)
        plsc.subcore_barrier()

        def body(g, _):
            pltpu.sync_copy(ones_v, hist_sh.at[col_v.at[g]], add=True)
            return 0
        lax.fori_loop(0, nch, body, 0)
        plsc.subcore_barrier()

        pltpu.sync_copy(hist_sh.at[pl.ds(sub * zr, zr)], z_v)
        pltpu.sync_copy(z_v, out_hbm.at[pl.ds(core * n + sub * zr, zr)])
        if tail:
            @pl.when(sub == 0)
            def _():
                pltpu.sync_copy(hist_sh.at[pl.ds(zr * NS, tail)],
                                z_v.at[pl.ds(0, tail)])
                pltpu.sync_copy(z_v.at[pl.ds(0, tail)],
                                out_hbm.at[pl.ds(core * n + zr * NS, tail)])

    return hist_k


def _make_scatter(n, nch, d):
    assert nch % 4 == 0
    hch = nch // 2
    zr = 32
    cr = (n // NS) & ~7
    nz = cr // zr
    zrem = cr - nz * zr
    tail = n - cr * NS
    mesh = plsc.VectorSubcoreMesh(core_axis_name="c", subcore_axis_name="s")

    @functools.partial(
        pl.kernel, mesh=mesh,
        out_type=jax.ShapeDtypeStruct((NC, n, d), jnp.float32),
        scratch_types=[
            pltpu.VMEM_SHARED((n, d), jnp.float32),
            pltpu.VMEM((hch, K), jnp.int32),
            pltpu.VMEM((hch, K), jnp.int32),
            pltpu.VMEM((2, K, d), jnp.float32),
            pltpu.VMEM((zr, d), jnp.float32),
            pltpu.SemaphoreType.DMA,
            pltpu.SemaphoreType.DMA,
        ],
    )
    def scat_k(xs_hbm, row_hbm, col_hbm, out_hbm,
               acc_sh, row_v, col_v, bufs, z_v, sem0, sem1):
        core = lax.axis_index("c")
        sub = lax.axis_index("s")
        wid = sub * NC + core
        sems = (sem0, sem1)

        def zfill(r, _):
            for l in range(d // 16):
                z_v[r, pl.ds(l * 16, 16)] = jnp.zeros((16,), jnp.float32)
            return 0
        lax.fori_loop(0, zr, zfill, 0)

        def zcopy(j, _):
            pltpu.sync_copy(z_v, acc_sh.at[pl.ds(sub * cr + j * zr, zr)])
            return 0
        lax.fori_loop(0, nz, zcopy, 0)
        if zrem:
            pltpu.sync_copy(z_v.at[pl.ds(0, zrem)],
                            acc_sh.at[pl.ds(sub * cr + nz * zr, zrem)])
        if tail:
            @pl.when(sub == 0)
            def _():
                pltpu.sync_copy(z_v.at[pl.ds(0, tail)],
                                acc_sh.at[pl.ds(cr * NS, tail)])
        plsc.subcore_barrier()

        for h in range(2):
            pltpu.sync_copy(row_hbm.at[wid, pl.ds(h * hch, hch)], row_v)
            pltpu.sync_copy(col_hbm.at[wid, pl.ds(h * hch, hch)], col_v)

            pltpu.async_copy(xs_hbm.at[row_v.at[0]], bufs.at[0], sem0)
            pltpu.async_copy(xs_hbm.at[row_v.at[1]], bufs.at[1], sem1)

            def body(o, _):
                for b in range(2):
                    g = o * 2 + b
                    pltpu.make_async_copy(
                        xs_hbm.at[row_v.at[g]], bufs.at[b], sems[b]).wait()
                    pltpu.sync_copy(bufs.at[b], acc_sh.at[col_v.at[g]],
                                    add=True)

                    @pl.when(g + 2 < hch)
                    def _():
                        pltpu.async_copy(
                            xs_hbm.at[row_v.at[g + 2]], bufs.at[b], sems[b])
                return 0
            lax.fori_loop(0, hch // 2, body, 0)
        plsc.subcore_barrier()

        pltpu.sync_copy(acc_sh.at[pl.ds(sub * cr, cr)],
                        out_hbm.at[core, pl.ds(sub * cr, cr)])
        if tail:
            @pl.when(sub == 0)
            def _():
                pltpu.sync_copy(acc_sh.at[pl.ds(cr * NS, tail)],
                                out_hbm.at[core, pl.ds(cr * NS, tail)])

    return scat_k


def _make_prep(n, d, bn, n_pad, pad):
    nb = n_pad // bn

    def prep_k(x_ref, hist_ref, wl_ref, fcw_ref, w1_ref, b1_ref, w2_ref,
               b2_ref, alpha_ref, fcb_ref, xs_ref, s_ref, wcs_ref, rv_ref,
               acc):
        i = pl.program_id(0)
        gri = i * bn + lax.broadcasted_iota(jnp.int32, (bn, 1), 0)
        deg = (jnp.sum(hist_ref[0], axis=1, keepdims=True)
               - jnp.where(gri < pad, 1.0, 0.0))
        s = jnp.where(deg > 0.0, lax.rsqrt(jnp.maximum(deg, 1e-30)), 0.0)
        valid = gri < n
        xb = jnp.where(valid, x_ref[...], 0.0)
        xs_ref[...] = jnp.where(valid, xb * s, 0.0)
        s_ref[...] = s

        psum = jnp.sum(xb, axis=0, keepdims=True)

        @pl.when(i == 0)
        def _():
            acc[0:1, :] = psum

        @pl.when(i > 0)
        def _():
            acc[0:1, :] = acc[0:1, :] + psum

        @pl.when(i == nb - 1)
        def _():
            mean = acc[0:1, :] * (1.0 / n)
            sig = jax.nn.sigmoid(alpha_ref[...])
            cdims = (((1,), (1,)), ((), ()))
            h = jnp.maximum(
                lax.dot_general(mean, w1_ref[...], cdims,
                                preferred_element_type=jnp.float32)
                + b1_ref[...], 0.0)
            g = lax.dot_general(h, w2_ref[...], cdims,
                                preferred_element_type=jnp.float32) + b2_ref[...]
            gf = lax.dot_general(g, fcw_ref[...], cdims,
                                 preferred_element_type=jnp.float32)
            rv_ref[...] = (1.0 - sig) * gf + fcb_ref[...]
            wcs_ref[...] = sig * lax.dot_general(
                wl_ref[...], fcw_ref[...], cdims,
                preferred_element_type=jnp.float32)

    full = lambda i: (0, 0)
    return pl.pallas_call(
        prep_k,
        grid=(nb,),
        in_specs=[
            pl.BlockSpec((bn, d), lambda i: (i, 0)),
            pl.BlockSpec((1, bn, 2), lambda i: (i, 0, 0)),
            pl.BlockSpec((d, d), full),
            pl.BlockSpec((d, d), full),
            pl.BlockSpec((d, d), full),
            pl.BlockSpec((1, d), full),
            pl.BlockSpec((d, d), full),
            pl.BlockSpec((1, d), full),
            pl.BlockSpec((1, 1), full),
            pl.BlockSpec((1, d), full),
        ],
        out_specs=[
            pl.BlockSpec((bn, d), lambda i: (i, 0)),
            pl.BlockSpec((bn, 1), lambda i: (i, 0)),
            pl.BlockSpec((d, d), full),
            pl.BlockSpec((1, d), full),
        ],
        out_shape=[
            jax.ShapeDtypeStruct((n_pad, d), jnp.float32),
            jax.ShapeDtypeStruct((n_pad, 1), jnp.float32),
            jax.ShapeDtypeStruct((d, d), jnp.float32),
            jax.ShapeDtypeStruct((1, d), jnp.float32),
        ],
        scratch_shapes=[pltpu.VMEM((8, d), jnp.float32)],
    )


def _make_final(n, d, bn):
    nb = n // bn

    def fin_k(acc_ref, s_ref, wcs_ref, rv_ref, out_ref):
        a = (acc_ref[0] + acc_ref[1]) * s_ref[...]
        out_ref[...] = jnp.dot(a, wcs_ref[...],
                               preferred_element_type=jnp.float32) + rv_ref[...]

    full = lambda i: (0, 0)
    return pl.pallas_call(
        fin_k,
        grid=(nb,),
        in_specs=[
            pl.BlockSpec((NC, bn, d), lambda i: (0, i, 0)),
            pl.BlockSpec((bn, 1), lambda i: (i, 0)),
            pl.BlockSpec((d, d), full),
            pl.BlockSpec((1, d), full),
        ],
        out_specs=pl.BlockSpec((bn, d), lambda i: (i, 0)),
        out_shape=jax.ShapeDtypeStruct((n, d), jnp.float32),
    )


def kernel(x, edge_index, weight_local, w1, b1, w2, b2, alpha, fc_w, fc_b):
    n, d = x.shape
    e = edge_index.shape[1]
    bn = 1000
    bp = 840
    n_pad = -(-n // bp) * bp

    row = edge_index[0].astype(jnp.int32)
    col = edge_index[1].astype(jnp.int32)

    nch = -(-e // (NW * K))
    nch += (-nch) % 4
    e_pad = NW * K * nch
    pad = e_pad - e
    assert pad < n and n + 8 <= n_pad
    if pad:
        ar = jnp.arange(pad, dtype=jnp.int32)
        row_p = jnp.concatenate([row, n + (ar % 8)])
        col_p = jnp.concatenate([col, ar % n])
    else:
        row_p, col_p = row, col
    row3 = row_p.reshape(NW, nch, K)
    col3 = col_p.reshape(NW, nch, K)

    hists = _make_hist(n, nch)(col3).reshape(NC, n)
    histt = jnp.concatenate(
        [jnp.transpose(hists),
         jnp.zeros((n_pad - n, NC), jnp.float32)]).reshape(-1, bp, NC)

    xs, s, wcs, rv = _make_prep(n, d, bp, n_pad, pad)(
        x, histt, weight_local, fc_w, w1, b1.reshape(1, d), w2,
        b2.reshape(1, d), alpha.reshape(1, 1), fc_b.reshape(1, d))

    accp = _make_scatter(n, nch, d)(xs, row3, col3)

    return _make_final(n, d, bn)(accp, s, wcs, rv)


# speedup vs baseline: 28.4109x; 1.0178x over previous
import functools

import jax
import jax.numpy as jnp
from jax import lax
from jax.experimental import pallas as pl
from jax.experimental.pallas import tpu as pltpu
from jax.experimental.pallas import tpu_sc as plsc

NC = 2
NS = 16
NW = NC * NS
K = 128
TRASH = 16


def _make_hist(n, nch):
    zr = (n // NS) & ~7
    tail = n - zr * NS
    mesh = plsc.VectorSubcoreMesh(core_axis_name="c", subcore_axis_name="s")

    @functools.partial(
        pl.kernel, mesh=mesh,
        out_type=jax.ShapeDtypeStruct((NC * n,), jnp.float32),
        scratch_types=[
            pltpu.VMEM_SHARED((n,), jnp.float32),
            pltpu.VMEM((nch, K), jnp.int32),
            pltpu.VMEM((K,), jnp.float32),
            pltpu.VMEM((zr,), jnp.float32),
        ],
    )
    def hist_k(col_hbm, out_hbm, hist_sh, col_v, ones_v, z_v):
        core = lax.axis_index("c")
        sub = lax.axis_index("s")
        wid = sub * NC + core

        def fill_ones(i, _):
            ones_v[pl.ds(i * 16, 16)] = jnp.ones((16,), jnp.float32)
            return 0
        lax.fori_loop(0, K // 16, fill_ones, 0)

        def fill_zero(i, _):
            z_v[pl.ds(i * 16, 16)] = jnp.zeros((16,), jnp.float32)
            return 0
        lax.fori_loop(0, zr // 16, fill_zero, 0)

        pltpu.sync_copy(z_v, hist_sh.at[pl.ds(sub * zr, zr)])
        if tail:
            @pl.when(sub == 0)
            def _():
                pltpu.sync_copy(z_v.at[pl.ds(0, tail)],
                                hist_sh.at[pl.ds(zr * NS, tail)])
        pltpu.sync_copy(col_hbm.at[wid], col_v)
        plsc.subcore_barrier()

        def body(g, _):
            pltpu.sync_copy(ones_v, hist_sh.at[col_v.at[g]], add=True)
            return 0
        lax.fori_loop(0, nch, body, 0)
        plsc.subcore_barrier()

        pltpu.sync_copy(hist_sh.at[pl.ds(sub * zr, zr)], z_v)
        pltpu.sync_copy(z_v, out_hbm.at[pl.ds(core * n + sub * zr, zr)])
        if tail:
            @pl.when(sub == 0)
            def _():
                pltpu.sync_copy(hist_sh.at[pl.ds(zr * NS, tail)],
                                z_v.at[pl.ds(0, tail)])
                pltpu.sync_copy(z_v.at[pl.ds(0, tail)],
                                out_hbm.at[pl.ds(core * n + zr * NS, tail)])

    return hist_k


def _make_scatter(n, nch, d):
    NB = 2
    assert nch % (2 * NB) == 0
    hch = nch // 2
    zr = 32
    cr = (n // NS) & ~7
    nz = cr // zr
    zrem = cr - nz * zr
    tail = n - cr * NS
    mesh = plsc.VectorSubcoreMesh(core_axis_name="c", subcore_axis_name="s")

    @functools.partial(
        pl.kernel, mesh=mesh,
        out_type=jax.ShapeDtypeStruct((NC, n, d), jnp.float32),
        scratch_types=[
            pltpu.VMEM_SHARED((n, d), jnp.float32),
            pltpu.VMEM((hch, K), jnp.int32),
            pltpu.VMEM((hch, K), jnp.int32),
            pltpu.VMEM((NB, K, d), jnp.float32),
            pltpu.VMEM((zr, d), jnp.float32),
            [pltpu.SemaphoreType.DMA] * NB,
        ],
    )
    def scat_k(xs_hbm, row_hbm, col_hbm, out_hbm,
               acc_sh, row_v, col_v, bufs, z_v, sems):
        core = lax.axis_index("c")
        sub = lax.axis_index("s")
        wid = sub * NC + core

        def zfill(r, _):
            for l in range(d // 16):
                z_v[r, pl.ds(l * 16, 16)] = jnp.zeros((16,), jnp.float32)
            return 0
        lax.fori_loop(0, zr, zfill, 0)

        def zcopy(j, _):
            pltpu.async_copy(z_v, acc_sh.at[pl.ds(sub * cr + j * zr, zr)],
                             sems[0])
            return 0
        lax.fori_loop(0, nz, zcopy, 0)
        if zrem:
            pltpu.sync_copy(z_v.at[pl.ds(0, zrem)],
                            acc_sh.at[pl.ds(sub * cr + nz * zr, zrem)])
        if tail:
            @pl.when(sub == 0)
            def _():
                pltpu.sync_copy(z_v.at[pl.ds(0, tail)],
                                acc_sh.at[pl.ds(cr * NS, tail)])
        pltpu.sync_copy(row_hbm.at[wid, pl.ds(0, hch)], row_v)
        pltpu.sync_copy(col_hbm.at[wid, pl.ds(0, hch)], col_v)

        def zdrain(j, _):
            pltpu.make_async_copy(
                z_v, acc_sh.at[pl.ds(sub * cr, zr)], sems[0]).wait()
            return 0
        lax.fori_loop(0, nz, zdrain, 0)
        plsc.subcore_barrier()

        for h in range(2):
            if h:
                pltpu.sync_copy(row_hbm.at[wid, pl.ds(h * hch, hch)], row_v)
                pltpu.sync_copy(col_hbm.at[wid, pl.ds(h * hch, hch)], col_v)

            for b in range(NB):
                pltpu.async_copy(xs_hbm.at[row_v.at[b]], bufs.at[b], sems[b])

            def body(o, _):
                for b in range(NB):
                    g = o * NB + b
                    pltpu.make_async_copy(
                        xs_hbm.at[row_v.at[g]], bufs.at[b], sems[b]).wait()
                    pltpu.sync_copy(bufs.at[b], acc_sh.at[col_v.at[g]],
                                    add=True)

                    @pl.when(g + NB < hch)
                    def _():
                        pltpu.async_copy(
                            xs_hbm.at[row_v.at[g + NB]], bufs.at[b], sems[b])
                return 0
            lax.fori_loop(0, hch // NB, body, 0)
        plsc.subcore_barrier()

        pltpu.sync_copy(acc_sh.at[pl.ds(sub * cr, cr)],
                        out_hbm.at[core, pl.ds(sub * cr, cr)])
        if tail:
            @pl.when(sub == 0)
            def _():
                pltpu.sync_copy(acc_sh.at[pl.ds(cr * NS, tail)],
                                out_hbm.at[core, pl.ds(cr * NS, tail)])

    return scat_k


def _make_prep(n, d, bn, n_pad, pad):
    nb = n_pad // bn

    def prep_k(x_ref, hist_ref, wl_ref, fcw_ref, w1_ref, b1_ref, w2_ref,
               b2_ref, alpha_ref, fcb_ref, xs_ref, s_ref, wcs_ref, rv_ref,
               acc):
        i = pl.program_id(0)
        gri = i * bn + lax.broadcasted_iota(jnp.int32, (bn, 1), 0)
        deg = (jnp.sum(hist_ref[0], axis=1, keepdims=True)
               - jnp.where(gri < pad, 1.0, 0.0))
        s = jnp.where(deg > 0.0, lax.rsqrt(jnp.maximum(deg, 1e-30)), 0.0)
        valid = gri < n
        xb = jnp.where(valid, x_ref[...], 0.0)
        xs_ref[...] = jnp.where(valid, xb * s, 0.0)
        s_ref[...] = s

        psum = jnp.sum(xb, axis=0, keepdims=True)

        @pl.when(i == 0)
        def _():
            acc[0:1, :] = psum

        @pl.when(i > 0)
        def _():
            acc[0:1, :] = acc[0:1, :] + psum

        @pl.when(i == nb - 1)
        def _():
            mean = acc[0:1, :] * (1.0 / n)
            sig = jax.nn.sigmoid(alpha_ref[...])
            cdims = (((1,), (1,)), ((), ()))
            h = jnp.maximum(
                lax.dot_general(mean, w1_ref[...], cdims,
                                preferred_element_type=jnp.float32)
                + b1_ref[...], 0.0)
            g = lax.dot_general(h, w2_ref[...], cdims,
                                preferred_element_type=jnp.float32) + b2_ref[...]
            gf = lax.dot_general(g, fcw_ref[...], cdims,
                                 preferred_element_type=jnp.float32)
            rv_ref[...] = (1.0 - sig) * gf + fcb_ref[...]
            wcs_ref[...] = sig * lax.dot_general(
                wl_ref[...], fcw_ref[...], cdims,
                preferred_element_type=jnp.float32)

    full = lambda i: (0, 0)
    return pl.pallas_call(
        prep_k,
        grid=(nb,),
        in_specs=[
            pl.BlockSpec((bn, d), lambda i: (i, 0)),
            pl.BlockSpec((1, bn, 2), lambda i: (i, 0, 0)),
            pl.BlockSpec((d, d), full),
            pl.BlockSpec((d, d), full),
            pl.BlockSpec((d, d), full),
            pl.BlockSpec((1, d), full),
            pl.BlockSpec((d, d), full),
            pl.BlockSpec((1, d), full),
            pl.BlockSpec((1, 1), full),
            pl.BlockSpec((1, d), full),
        ],
        out_specs=[
            pl.BlockSpec((bn, d), lambda i: (i, 0)),
            pl.BlockSpec((bn, 1), lambda i: (i, 0)),
            pl.BlockSpec((d, d), full),
            pl.BlockSpec((1, d), full),
        ],
        out_shape=[
            jax.ShapeDtypeStruct((n_pad, d), jnp.float32),
            jax.ShapeDtypeStruct((n_pad, 1), jnp.float32),
            jax.ShapeDtypeStruct((d, d), jnp.float32),
            jax.ShapeDtypeStruct((1, d), jnp.float32),
        ],
        scratch_shapes=[pltpu.VMEM((8, d), jnp.float32)],
    )


def _make_final(n, d, bn):
    nb = n // bn

    def fin_k(acc_ref, s_ref, wcs_ref, rv_ref, out_ref):
        a = (acc_ref[0] + acc_ref[1]) * s_ref[...]
        out_ref[...] = jnp.dot(a, wcs_ref[...],
                               preferred_element_type=jnp.float32) + rv_ref[...]

    full = lambda i: (0, 0)
    return pl.pallas_call(
        fin_k,
        grid=(nb,),
        in_specs=[
            pl.BlockSpec((NC, bn, d), lambda i: (0, i, 0)),
            pl.BlockSpec((bn, 1), lambda i: (i, 0)),
            pl.BlockSpec((d, d), full),
            pl.BlockSpec((1, d), full),
        ],
        out_specs=pl.BlockSpec((bn, d), lambda i: (i, 0)),
        out_shape=jax.ShapeDtypeStruct((n, d), jnp.float32),
    )


def kernel(x, edge_index, weight_local, w1, b1, w2, b2, alpha, fc_w, fc_b):
    n, d = x.shape
    e = edge_index.shape[1]
    bn = 1000
    bp = 1008
    n_pad = -(-n // bp) * bp

    row = edge_index[0].astype(jnp.int32)
    col = edge_index[1].astype(jnp.int32)

    nch = -(-e // (NW * K))
    nch += (-nch) % 4
    e_pad = NW * K * nch
    pad = e_pad - e
    assert pad < n and n + 8 <= n_pad
    if pad:
        ar = jnp.arange(pad, dtype=jnp.int32)
        row_p = jnp.concatenate([row, n + (ar % 8)])
        col_p = jnp.concatenate([col, ar % n])
    else:
        row_p, col_p = row, col
    row3 = row_p.reshape(NW, nch, K)
    col3 = col_p.reshape(NW, nch, K)

    hists = _make_hist(n, nch)(col3).reshape(NC, n)
    histt = jnp.concatenate(
        [jnp.transpose(hists),
         jnp.zeros((n_pad - n, NC), jnp.float32)]).reshape(-1, bp, NC)

    xs, s, wcs, rv = _make_prep(n, d, bp, n_pad, pad)(
        x, histt, weight_local, fc_w, w1, b1.reshape(1, d), w2,
        b2.reshape(1, d), alpha.reshape(1, 1), fc_b.reshape(1, d))

    accp = _make_scatter(n, nch, d)(xs, row3, col3)

    return _make_final(n, d, bn)(accp, s, wcs, rv)


# speedup vs baseline: 29.0017x; 1.0208x over previous
import functools

import jax
import jax.numpy as jnp
from jax import lax
from jax.experimental import pallas as pl
from jax.experimental.pallas import tpu as pltpu
from jax.experimental.pallas import tpu_sc as plsc

NC = 2
NS = 16
NW = NC * NS
K = 128
TRASH = 16


def _make_hist(n, nch):
    zr = (n // NS) & ~7
    tail = n - zr * NS
    mesh = plsc.VectorSubcoreMesh(core_axis_name="c", subcore_axis_name="s")

    @functools.partial(
        pl.kernel, mesh=mesh,
        out_type=jax.ShapeDtypeStruct((NC * n,), jnp.float32),
        scratch_types=[
            pltpu.VMEM_SHARED((n,), jnp.float32),
            pltpu.VMEM((nch, K), jnp.int32),
            pltpu.VMEM((K,), jnp.float32),
            pltpu.VMEM((zr,), jnp.float32),
            pltpu.SemaphoreType.DMA,
        ],
    )
    def hist_k(col_hbm, out_hbm, hist_sh, col_v, ones_v, z_v, hsem):
        core = lax.axis_index("c")
        sub = lax.axis_index("s")
        wid = sub * NC + core

        def fill_ones(i, _):
            ones_v[pl.ds(i * 16, 16)] = jnp.ones((16,), jnp.float32)
            return 0
        lax.fori_loop(0, K // 16, fill_ones, 0)

        def fill_zero(i, _):
            z_v[pl.ds(i * 16, 16)] = jnp.zeros((16,), jnp.float32)
            return 0
        lax.fori_loop(0, zr // 16, fill_zero, 0)

        pltpu.sync_copy(z_v, hist_sh.at[pl.ds(sub * zr, zr)])
        if tail:
            @pl.when(sub == 0)
            def _():
                pltpu.sync_copy(z_v.at[pl.ds(0, tail)],
                                hist_sh.at[pl.ds(zr * NS, tail)])
        pltpu.sync_copy(col_hbm.at[wid], col_v)
        plsc.subcore_barrier()

        def body(o, _):
            for q in range(4):
                pltpu.async_copy(ones_v, hist_sh.at[col_v.at[o * 4 + q]],
                                 hsem, add=True)
            for q in range(4):
                pltpu.make_async_copy(
                    ones_v, hist_sh.at[col_v.at[o * 4]], hsem).wait()
            return 0
        lax.fori_loop(0, nch // 4, body, 0)
        rem = nch % 4
        for q in range(rem):
            pltpu.sync_copy(ones_v, hist_sh.at[col_v.at[nch - rem + q]],
                            add=True)
        plsc.subcore_barrier()

        pltpu.sync_copy(hist_sh.at[pl.ds(sub * zr, zr)], z_v)
        pltpu.sync_copy(z_v, out_hbm.at[pl.ds(core * n + sub * zr, zr)])
        if tail:
            @pl.when(sub == 0)
            def _():
                pltpu.sync_copy(hist_sh.at[pl.ds(zr * NS, tail)],
                                z_v.at[pl.ds(0, tail)])
                pltpu.sync_copy(z_v.at[pl.ds(0, tail)],
                                out_hbm.at[pl.ds(core * n + zr * NS, tail)])

    return hist_k


def _make_scatter(n, nch, d):
    NB = 2
    assert nch % (2 * NB) == 0
    hch = nch // 2
    zr = 32
    cr = (n // NS) & ~7
    nz = cr // zr
    zrem = cr - nz * zr
    tail = n - cr * NS
    mesh = plsc.VectorSubcoreMesh(core_axis_name="c", subcore_axis_name="s")

    @functools.partial(
        pl.kernel, mesh=mesh,
        out_type=jax.ShapeDtypeStruct((NC, n, d), jnp.float32),
        scratch_types=[
            pltpu.VMEM_SHARED((n, d), jnp.float32),
            pltpu.VMEM((hch, K), jnp.int32),
            pltpu.VMEM((hch, K), jnp.int32),
            pltpu.VMEM((NB, K, d), jnp.float32),
            pltpu.VMEM((zr, d), jnp.float32),
            [pltpu.SemaphoreType.DMA] * NB,
        ],
    )
    def scat_k(xs_hbm, row_hbm, col_hbm, out_hbm,
               acc_sh, row_v, col_v, bufs, z_v, sems):
        core = lax.axis_index("c")
        sub = lax.axis_index("s")
        wid = sub * NC + core

        def zfill(r, _):
            for l in range(d // 16):
                z_v[r, pl.ds(l * 16, 16)] = jnp.zeros((16,), jnp.float32)
            return 0
        lax.fori_loop(0, zr, zfill, 0)

        def zcopy(j, _):
            pltpu.async_copy(z_v, acc_sh.at[pl.ds(sub * cr + j * zr, zr)],
                             sems[0])
            return 0
        lax.fori_loop(0, nz, zcopy, 0)
        if zrem:
            pltpu.sync_copy(z_v.at[pl.ds(0, zrem)],
                            acc_sh.at[pl.ds(sub * cr + nz * zr, zrem)])
        if tail:
            @pl.when(sub == 0)
            def _():
                pltpu.sync_copy(z_v.at[pl.ds(0, tail)],
                                acc_sh.at[pl.ds(cr * NS, tail)])
        pltpu.sync_copy(row_hbm.at[wid, pl.ds(0, hch)], row_v)
        pltpu.sync_copy(col_hbm.at[wid, pl.ds(0, hch)], col_v)

        def zdrain(j, _):
            pltpu.make_async_copy(
                z_v, acc_sh.at[pl.ds(sub * cr, zr)], sems[0]).wait()
            return 0
        lax.fori_loop(0, nz, zdrain, 0)
        plsc.subcore_barrier()

        for h in range(2):
            if h:
                pltpu.sync_copy(row_hbm.at[wid, pl.ds(h * hch, hch)], row_v)
                pltpu.sync_copy(col_hbm.at[wid, pl.ds(h * hch, hch)], col_v)

            for b in range(NB):
                pltpu.async_copy(xs_hbm.at[row_v.at[b]], bufs.at[b], sems[b])

            def body(o, _):
                for b in range(NB):
                    g = o * NB + b
                    pltpu.make_async_copy(
                        xs_hbm.at[row_v.at[g]], bufs.at[b], sems[b]).wait()
                    pltpu.sync_copy(bufs.at[b], acc_sh.at[col_v.at[g]],
                                    add=True)

                    @pl.when(g + NB < hch)
                    def _():
                        pltpu.async_copy(
                            xs_hbm.at[row_v.at[g + NB]], bufs.at[b], sems[b])
                return 0
            lax.fori_loop(0, hch // NB, body, 0)
        plsc.subcore_barrier()

        pltpu.sync_copy(acc_sh.at[pl.ds(sub * cr, cr)],
                        out_hbm.at[core, pl.ds(sub * cr, cr)])
        if tail:
            @pl.when(sub == 0)
            def _():
                pltpu.sync_copy(acc_sh.at[pl.ds(cr * NS, tail)],
                                out_hbm.at[core, pl.ds(cr * NS, tail)])

    return scat_k


def _make_prep(n, d, bn, n_pad, pad):
    nb = n_pad // bn

    def prep_k(x_ref, hist_ref, wl_ref, fcw_ref, w1_ref, b1_ref, w2_ref,
               b2_ref, alpha_ref, fcb_ref, xs_ref, s_ref, wcs_ref, rv_ref,
               acc):
        i = pl.program_id(0)
        gri = i * bn + lax.broadcasted_iota(jnp.int32, (bn, 1), 0)
        deg = (jnp.sum(hist_ref[0], axis=1, keepdims=True)
               - jnp.where(gri < pad, 1.0, 0.0))
        s = jnp.where(deg > 0.0, lax.rsqrt(jnp.maximum(deg, 1e-30)), 0.0)
        valid = gri < n
        xb = jnp.where(valid, x_ref[...], 0.0)
        xs_ref[...] = jnp.where(valid, xb * s, 0.0)
        s_ref[...] = s

        psum = jnp.sum(xb, axis=0, keepdims=True)

        @pl.when(i == 0)
        def _():
            acc[0:1, :] = psum

        @pl.when(i > 0)
        def _():
            acc[0:1, :] = acc[0:1, :] + psum

        @pl.when(i == nb - 1)
        def _():
            mean = acc[0:1, :] * (1.0 / n)
            sig = jax.nn.sigmoid(alpha_ref[...])
            cdims = (((1,), (1,)), ((), ()))
            h = jnp.maximum(
                lax.dot_general(mean, w1_ref[...], cdims,
                                preferred_element_type=jnp.float32)
                + b1_ref[...], 0.0)
            g = lax.dot_general(h, w2_ref[...], cdims,
                                preferred_element_type=jnp.float32) + b2_ref[...]
            gf = lax.dot_general(g, fcw_ref[...], cdims,
                                 preferred_element_type=jnp.float32)
            rv_ref[...] = (1.0 - sig) * gf + fcb_ref[...]
            wcs_ref[...] = sig * lax.dot_general(
                wl_ref[...], fcw_ref[...], cdims,
                preferred_element_type=jnp.float32)

    full = lambda i: (0, 0)
    return pl.pallas_call(
        prep_k,
        grid=(nb,),
        in_specs=[
            pl.BlockSpec((bn, d), lambda i: (i, 0)),
            pl.BlockSpec((1, bn, 2), lambda i: (i, 0, 0)),
            pl.BlockSpec((d, d), full),
            pl.BlockSpec((d, d), full),
            pl.BlockSpec((d, d), full),
            pl.BlockSpec((1, d), full),
            pl.BlockSpec((d, d), full),
            pl.BlockSpec((1, d), full),
            pl.BlockSpec((1, 1), full),
            pl.BlockSpec((1, d), full),
        ],
        out_specs=[
            pl.BlockSpec((bn, d), lambda i: (i, 0)),
            pl.BlockSpec((bn, 1), lambda i: (i, 0)),
            pl.BlockSpec((d, d), full),
            pl.BlockSpec((1, d), full),
        ],
        out_shape=[
            jax.ShapeDtypeStruct((n_pad, d), jnp.float32),
            jax.ShapeDtypeStruct((n_pad, 1), jnp.float32),
            jax.ShapeDtypeStruct((d, d), jnp.float32),
            jax.ShapeDtypeStruct((1, d), jnp.float32),
        ],
        scratch_shapes=[pltpu.VMEM((8, d), jnp.float32)],
    )


def _make_final(n, d, bn):
    nb = n // bn

    def fin_k(acc_ref, s_ref, wcs_ref, rv_ref, out_ref):
        a = (acc_ref[0] + acc_ref[1]) * s_ref[...]
        out_ref[...] = jnp.dot(a, wcs_ref[...],
                               preferred_element_type=jnp.float32) + rv_ref[...]

    full = lambda i: (0, 0)
    return pl.pallas_call(
        fin_k,
        grid=(nb,),
        in_specs=[
            pl.BlockSpec((NC, bn, d), lambda i: (0, i, 0)),
            pl.BlockSpec((bn, 1), lambda i: (i, 0)),
            pl.BlockSpec((d, d), full),
            pl.BlockSpec((1, d), full),
        ],
        out_specs=pl.BlockSpec((bn, d), lambda i: (i, 0)),
        out_shape=jax.ShapeDtypeStruct((n, d), jnp.float32),
    )


def kernel(x, edge_index, weight_local, w1, b1, w2, b2, alpha, fc_w, fc_b):
    n, d = x.shape
    e = edge_index.shape[1]
    bn = 1000
    bp = 1008
    n_pad = -(-n // bp) * bp

    row = edge_index[0].astype(jnp.int32)
    col = edge_index[1].astype(jnp.int32)

    nch = -(-e // (NW * K))
    nch += (-nch) % 4
    e_pad = NW * K * nch
    pad = e_pad - e
    assert pad < n and n + 8 <= n_pad
    if pad:
        ar = jnp.arange(pad, dtype=jnp.int32)
        row_p = jnp.concatenate([row, n + (ar % 8)])
        col_p = jnp.concatenate([col, ar % n])
    else:
        row_p, col_p = row, col
    row3 = row_p.reshape(NW, nch, K)
    col3 = col_p.reshape(NW, nch, K)

    hists = _make_hist(n, nch)(col3).reshape(NC, n)
    histt = jnp.concatenate(
        [jnp.transpose(hists),
         jnp.zeros((n_pad - n, NC), jnp.float32)]).reshape(-1, bp, NC)

    xs, s, wcs, rv = _make_prep(n, d, bp, n_pad, pad)(
        x, histt, weight_local, fc_w, w1, b1.reshape(1, d), w2,
        b2.reshape(1, d), alpha.reshape(1, 1), fc_b.reshape(1, d))

    accp = _make_scatter(n, nch, d)(xs, row3, col3)

    return _make_final(n, d, bn)(accp, s, wcs, rv)


# speedup vs baseline: 31.1096x; 1.0727x over previous
import functools

import jax
import jax.numpy as jnp
from jax import lax
from jax.experimental import pallas as pl
from jax.experimental.pallas import tpu as pltpu
from jax.experimental.pallas import tpu_sc as plsc

NC = 2
NS = 16
NW = NC * NS
K = 128
TRASH = 16


def _make_hist(n, nch):
    zr = (n // NS) & ~7
    tail = n - zr * NS
    mesh = plsc.VectorSubcoreMesh(core_axis_name="c", subcore_axis_name="s")

    @functools.partial(
        pl.kernel, mesh=mesh,
        out_type=jax.ShapeDtypeStruct((NC * n,), jnp.float32),
        scratch_types=[
            pltpu.VMEM_SHARED((n,), jnp.float32),
            pltpu.VMEM((nch, K), jnp.int32),
            pltpu.VMEM((K,), jnp.float32),
            pltpu.VMEM((zr,), jnp.float32),
            pltpu.SemaphoreType.DMA,
        ],
    )
    def hist_k(col_hbm, out_hbm, hist_sh, col_v, ones_v, z_v, hsem):
        core = lax.axis_index("c")
        sub = lax.axis_index("s")
        wid = sub * NC + core

        def fill_ones(i, _):
            ones_v[pl.ds(i * 16, 16)] = jnp.ones((16,), jnp.float32)
            return 0
        lax.fori_loop(0, K // 16, fill_ones, 0)

        def fill_zero(i, _):
            z_v[pl.ds(i * 16, 16)] = jnp.zeros((16,), jnp.float32)
            return 0
        lax.fori_loop(0, zr // 16, fill_zero, 0)

        pltpu.sync_copy(z_v, hist_sh.at[pl.ds(sub * zr, zr)])
        if tail:
            @pl.when(sub == 0)
            def _():
                pltpu.sync_copy(z_v.at[pl.ds(0, tail)],
                                hist_sh.at[pl.ds(zr * NS, tail)])
        pltpu.sync_copy(col_hbm.at[wid], col_v)
        plsc.subcore_barrier()

        def body(o, _):
            for q in range(4):
                pltpu.async_copy(ones_v, hist_sh.at[col_v.at[o * 4 + q]],
                                 hsem, add=True)
            for q in range(4):
                pltpu.make_async_copy(
                    ones_v, hist_sh.at[col_v.at[o * 4]], hsem).wait()
            return 0
        lax.fori_loop(0, nch // 4, body, 0)
        rem = nch % 4
        for q in range(rem):
            pltpu.sync_copy(ones_v, hist_sh.at[col_v.at[nch - rem + q]],
                            add=True)
        plsc.subcore_barrier()

        pltpu.sync_copy(hist_sh.at[pl.ds(sub * zr, zr)], z_v)
        pltpu.sync_copy(z_v, out_hbm.at[pl.ds(core * n + sub * zr, zr)])
        if tail:
            @pl.when(sub == 0)
            def _():
                pltpu.sync_copy(hist_sh.at[pl.ds(zr * NS, tail)],
                                z_v.at[pl.ds(0, tail)])
                pltpu.sync_copy(z_v.at[pl.ds(0, tail)],
                                out_hbm.at[pl.ds(core * n + zr * NS, tail)])

    return hist_k


def _make_scatter(n, nch, d):
    NB = 2
    assert nch % (2 * NB) == 0
    hch = nch // 2
    zr = 32
    cr = (n // NS) & ~7
    nz = cr // zr
    zrem = cr - nz * zr
    tail = n - cr * NS
    mesh = plsc.VectorSubcoreMesh(core_axis_name="c", subcore_axis_name="s")

    @functools.partial(
        pl.kernel, mesh=mesh,
        out_type=jax.ShapeDtypeStruct((NC, n, d), jnp.float32),
        scratch_types=[
            pltpu.VMEM_SHARED((n, d), jnp.float32),
            pltpu.VMEM((hch, K), jnp.int32),
            pltpu.VMEM((hch, K), jnp.int32),
            pltpu.VMEM((NB, K, d), jnp.float32),
            pltpu.VMEM((zr, d), jnp.float32),
            [pltpu.SemaphoreType.DMA] * NB,
        ],
    )
    def scat_k(xs_hbm, row_hbm, col_hbm, out_hbm,
               acc_sh, row_v, col_v, bufs, z_v, sems):
        core = lax.axis_index("c")
        sub = lax.axis_index("s")
        wid = sub * NC + core

        def zfill(r, _):
            for l in range(d // 16):
                z_v[r, pl.ds(l * 16, 16)] = jnp.zeros((16,), jnp.float32)
            return 0
        lax.fori_loop(0, zr, zfill, 0)

        def zcopy(j, _):
            pltpu.async_copy(z_v, acc_sh.at[pl.ds(sub * cr + j * zr, zr)],
                             sems[0])
            return 0
        lax.fori_loop(0, nz, zcopy, 0)
        if zrem:
            pltpu.sync_copy(z_v.at[pl.ds(0, zrem)],
                            acc_sh.at[pl.ds(sub * cr + nz * zr, zrem)])
        if tail:
            @pl.when(sub == 0)
            def _():
                pltpu.sync_copy(z_v.at[pl.ds(0, tail)],
                                acc_sh.at[pl.ds(cr * NS, tail)])
        pltpu.sync_copy(row_hbm.at[wid, pl.ds(0, hch)], row_v)
        pltpu.sync_copy(col_hbm.at[wid, pl.ds(0, hch)], col_v)

        def zdrain(j, _):
            pltpu.make_async_copy(
                z_v, acc_sh.at[pl.ds(sub * cr, zr)], sems[0]).wait()
            return 0
        lax.fori_loop(0, nz, zdrain, 0)
        plsc.subcore_barrier()

        for h in range(2):
            if h:
                pltpu.sync_copy(row_hbm.at[wid, pl.ds(h * hch, hch)], row_v)
                pltpu.sync_copy(col_hbm.at[wid, pl.ds(h * hch, hch)], col_v)

            for b in range(NB):
                pltpu.async_copy(xs_hbm.at[row_v.at[b]], bufs.at[b], sems[b])

            def body(o, _):
                for b in range(NB):
                    g = o * NB + b
                    pltpu.make_async_copy(
                        xs_hbm.at[row_v.at[g]], bufs.at[b], sems[b]).wait()
                    pltpu.sync_copy(bufs.at[b], acc_sh.at[col_v.at[g]],
                                    add=True)

                    @pl.when(g + NB < hch)
                    def _():
                        pltpu.async_copy(
                            xs_hbm.at[row_v.at[g + NB]], bufs.at[b], sems[b])
                return 0
            lax.fori_loop(0, hch // NB, body, 0)
        plsc.subcore_barrier()

        pltpu.sync_copy(acc_sh.at[pl.ds(sub * cr, cr)],
                        out_hbm.at[core, pl.ds(sub * cr, cr)])
        if tail:
            @pl.when(sub == 0)
            def _():
                pltpu.sync_copy(acc_sh.at[pl.ds(cr * NS, tail)],
                                out_hbm.at[core, pl.ds(cr * NS, tail)])

    return scat_k


def _make_prep(n, d, bn, n_pad, pad):
    nb = n_pad // bn

    def prep_k(x_ref, hist_ref, wl_ref, fcw_ref, w1_ref, b1_ref, w2_ref,
               b2_ref, alpha_ref, fcb_ref, xs_ref, s_ref, wcs_ref, rv_ref,
               acc):
        i = pl.program_id(0)
        gri = i * bn + lax.broadcasted_iota(jnp.int32, (bn, 1), 0)
        deg = (jnp.sum(hist_ref[0], axis=1, keepdims=True)
               - jnp.where(gri < pad, 1.0, 0.0))
        s = jnp.where(deg > 0.0, lax.rsqrt(jnp.maximum(deg, 1e-30)), 0.0)
        valid = gri < n
        xb = jnp.where(valid, x_ref[...], 0.0)
        xs_ref[...] = jnp.where(valid, xb * s, 0.0)
        s_ref[...] = s

        psum = jnp.sum(xb, axis=0, keepdims=True)

        @pl.when(i == 0)
        def _():
            acc[0:1, :] = psum

        @pl.when(i > 0)
        def _():
            acc[0:1, :] = acc[0:1, :] + psum

        @pl.when(i == nb - 1)
        def _():
            mean = acc[0:1, :] * (1.0 / n)
            sig = jax.nn.sigmoid(alpha_ref[...])
            cdims = (((1,), (1,)), ((), ()))
            h = jnp.maximum(
                lax.dot_general(mean, w1_ref[...], cdims,
                                preferred_element_type=jnp.float32)
                + b1_ref[...], 0.0)
            g = lax.dot_general(h, w2_ref[...], cdims,
                                preferred_element_type=jnp.float32) + b2_ref[...]
            gf = lax.dot_general(g, fcw_ref[...], cdims,
                                 preferred_element_type=jnp.float32)
            rv_ref[...] = (1.0 - sig) * gf + fcb_ref[...]
            wcs_ref[...] = sig * lax.dot_general(
                wl_ref[...], fcw_ref[...], cdims,
                preferred_element_type=jnp.float32)

    full = lambda i: (0, 0)
    return pl.pallas_call(
        prep_k,
        grid=(nb,),
        in_specs=[
            pl.BlockSpec((bn, d), lambda i: (i, 0)),
            pl.BlockSpec((1, bn, 2), lambda i: (i, 0, 0)),
            pl.BlockSpec((d, d), full),
            pl.BlockSpec((d, d), full),
            pl.BlockSpec((d, d), full),
            pl.BlockSpec((1, d), full),
            pl.BlockSpec((d, d), full),
            pl.BlockSpec((1, d), full),
            pl.BlockSpec((1, 1), full),
            pl.BlockSpec((1, d), full),
        ],
        out_specs=[
            pl.BlockSpec((bn, d), lambda i: (i, 0)),
            pl.BlockSpec((bn, 1), lambda i: (i, 0)),
            pl.BlockSpec((d, d), full),
            pl.BlockSpec((1, d), full),
        ],
        out_shape=[
            jax.ShapeDtypeStruct((n_pad, d), jnp.float32),
            jax.ShapeDtypeStruct((n_pad, 1), jnp.float32),
            jax.ShapeDtypeStruct((d, d), jnp.float32),
            jax.ShapeDtypeStruct((1, d), jnp.float32),
        ],
        scratch_shapes=[pltpu.VMEM((8, d), jnp.float32)],
    )


def _make_final(n, d, bn):
    nb = n // bn

    def fin_k(acc_ref, s_ref, wcs_ref, rv_ref, out_ref):
        a = (acc_ref[0] + acc_ref[1]) * s_ref[...]
        out_ref[...] = jnp.dot(a, wcs_ref[...],
                               preferred_element_type=jnp.float32) + rv_ref[...]

    full = lambda i: (0, 0)
    return pl.pallas_call(
        fin_k,
        grid=(nb,),
        in_specs=[
            pl.BlockSpec((NC, bn, d), lambda i: (0, i, 0)),
            pl.BlockSpec((bn, 1), lambda i: (i, 0)),
            pl.BlockSpec((d, d), full),
            pl.BlockSpec((1, d), full),
        ],
        out_specs=pl.BlockSpec((bn, d), lambda i: (i, 0)),
        out_shape=jax.ShapeDtypeStruct((n, d), jnp.float32),
    )


def kernel(x, edge_index, weight_local, w1, b1, w2, b2, alpha, fc_w, fc_b):
    n, d = x.shape
    e = edge_index.shape[1]
    bn = 1000
    bp = 1008
    n_pad = -(-n // bp) * bp

    row = edge_index[0].astype(jnp.int32)
    col = edge_index[1].astype(jnp.int32)

    nch = -(-e // (NW * K))
    nch += (-nch) % 4
    e_pad = NW * K * nch
    pad = e_pad - e
    assert pad < n and n + 8 <= n_pad
    if pad:
        ar = jnp.arange(pad, dtype=jnp.int32)
        row_p = jnp.concatenate([row, n + (ar % (n_pad - n))])
        col_p = jnp.concatenate([col, ar % n])
    else:
        row_p, col_p = row, col
    row3 = jnp.swapaxes(row_p.reshape(nch, NW, K), 0, 1)
    col3 = jnp.swapaxes(col_p.reshape(nch, NW, K), 0, 1)

    hists = _make_hist(n, nch)(col3).reshape(NC, n)
    histt = jnp.concatenate(
        [jnp.transpose(hists),
         jnp.zeros((n_pad - n, NC), jnp.float32)]).reshape(-1, bp, NC)

    xs, s, wcs, rv = _make_prep(n, d, bp, n_pad, pad)(
        x, histt, weight_local, fc_w, w1, b1.reshape(1, d), w2,
        b2.reshape(1, d), alpha.reshape(1, 1), fc_b.reshape(1, d))

    accp = _make_scatter(n, nch, d)(xs, row3, col3)

    return _make_final(n, d, bn)(accp, s, wcs, rv)


# speedup vs baseline: 31.2693x; 1.0051x over previous
import functools

import jax
import jax.numpy as jnp
from jax import lax
from jax.experimental import pallas as pl
from jax.experimental.pallas import tpu as pltpu
from jax.experimental.pallas import tpu_sc as plsc

NC = 2
NS = 16
NW = NC * NS
K = 128
TRASH = 16


def _make_hist(n, nch):
    zr = (n // NS) & ~7
    tail = n - zr * NS
    mesh = plsc.VectorSubcoreMesh(core_axis_name="c", subcore_axis_name="s")

    @functools.partial(
        pl.kernel, mesh=mesh,
        out_type=jax.ShapeDtypeStruct((NC * n,), jnp.float32),
        scratch_types=[
            pltpu.VMEM_SHARED((n,), jnp.float32),
            pltpu.VMEM((nch, K), jnp.int32),
            pltpu.VMEM((K,), jnp.float32),
            pltpu.VMEM((zr,), jnp.float32),
            pltpu.SemaphoreType.DMA,
        ],
    )
    def hist_k(col_hbm, out_hbm, hist_sh, col_v, ones_v, z_v, hsem):
        core = lax.axis_index("c")
        sub = lax.axis_index("s")
        wid = sub * NC + core

        def fill_ones(i, _):
            ones_v[pl.ds(i * 16, 16)] = jnp.ones((16,), jnp.float32)
            return 0
        lax.fori_loop(0, K // 16, fill_ones, 0)

        def fill_zero(i, _):
            z_v[pl.ds(i * 16, 16)] = jnp.zeros((16,), jnp.float32)
            return 0
        lax.fori_loop(0, zr // 16, fill_zero, 0)

        pltpu.sync_copy(z_v, hist_sh.at[pl.ds(sub * zr, zr)])
        if tail:
            @pl.when(sub == 0)
            def _():
                pltpu.sync_copy(z_v.at[pl.ds(0, tail)],
                                hist_sh.at[pl.ds(zr * NS, tail)])
        pltpu.sync_copy(col_hbm.at[wid], col_v)
        plsc.subcore_barrier()

        def body(o, _):
            for q in range(4):
                pltpu.async_copy(ones_v, hist_sh.at[col_v.at[o * 4 + q]],
                                 hsem, add=True)
            for q in range(4):
                pltpu.make_async_copy(
                    ones_v, hist_sh.at[col_v.at[o * 4]], hsem).wait()
            return 0
        lax.fori_loop(0, nch // 4, body, 0)
        rem = nch % 4
        for q in range(rem):
            pltpu.sync_copy(ones_v, hist_sh.at[col_v.at[nch - rem + q]],
                            add=True)
        plsc.subcore_barrier()

        pltpu.sync_copy(hist_sh.at[pl.ds(sub * zr, zr)], z_v)
        pltpu.sync_copy(z_v, out_hbm.at[pl.ds(core * n + sub * zr, zr)])
        if tail:
            @pl.when(sub == 0)
            def _():
                pltpu.sync_copy(hist_sh.at[pl.ds(zr * NS, tail)],
                                z_v.at[pl.ds(0, tail)])
                pltpu.sync_copy(z_v.at[pl.ds(0, tail)],
                                out_hbm.at[pl.ds(core * n + zr * NS, tail)])

    return hist_k


def _make_scatter(n, nch, d):
    NB = 2
    assert nch % (2 * NB) == 0
    hch = nch // 2
    zr = 32
    cr = (n // NS) & ~7
    nz = cr // zr
    zrem = cr - nz * zr
    tail = n - cr * NS
    mesh = plsc.VectorSubcoreMesh(core_axis_name="c", subcore_axis_name="s")

    @functools.partial(
        pl.kernel, mesh=mesh,
        out_type=jax.ShapeDtypeStruct((NC, n, d), jnp.float32),
        scratch_types=[
            pltpu.VMEM_SHARED((n, d), jnp.float32),
            pltpu.VMEM((hch, K), jnp.int32),
            pltpu.VMEM((hch, K), jnp.int32),
            pltpu.VMEM((NB, K, d), jnp.float32),
            pltpu.VMEM((zr, d), jnp.float32),
            [pltpu.SemaphoreType.DMA] * NB,
        ],
    )
    def scat_k(xs_hbm, row_hbm, col_hbm, out_hbm,
               acc_sh, row_v, col_v, bufs, z_v, sems):
        core = lax.axis_index("c")
        sub = lax.axis_index("s")
        wid = sub * NC + core

        def zfill(r, _):
            for l in range(d // 16):
                z_v[r, pl.ds(l * 16, 16)] = jnp.zeros((16,), jnp.float32)
            return 0
        lax.fori_loop(0, zr, zfill, 0)

        def zcopy(j, _):
            pltpu.async_copy(z_v, acc_sh.at[pl.ds(sub * cr + j * zr, zr)],
                             sems[0])
            return 0
        lax.fori_loop(0, nz, zcopy, 0)
        if zrem:
            pltpu.sync_copy(z_v.at[pl.ds(0, zrem)],
                            acc_sh.at[pl.ds(sub * cr + nz * zr, zrem)])
        if tail:
            @pl.when(sub == 0)
            def _():
                pltpu.sync_copy(z_v.at[pl.ds(0, tail)],
                                acc_sh.at[pl.ds(cr * NS, tail)])
        pltpu.sync_copy(row_hbm.at[wid, pl.ds(0, hch)], row_v)
        pltpu.sync_copy(col_hbm.at[wid, pl.ds(0, hch)], col_v)

        def zdrain(j, _):
            pltpu.make_async_copy(
                z_v, acc_sh.at[pl.ds(sub * cr, zr)], sems[0]).wait()
            return 0
        lax.fori_loop(0, nz, zdrain, 0)
        plsc.subcore_barrier()

        for h in range(2):
            if h:
                pltpu.sync_copy(row_hbm.at[wid, pl.ds(h * hch, hch)], row_v)
                pltpu.sync_copy(col_hbm.at[wid, pl.ds(h * hch, hch)], col_v)

            for b in range(NB):
                pltpu.async_copy(xs_hbm.at[row_v.at[b]], bufs.at[b], sems[b])

            def body(o, _):
                for b in range(NB):
                    g = o * NB + b
                    pltpu.make_async_copy(
                        xs_hbm.at[row_v.at[g]], bufs.at[b], sems[b]).wait()
                    pltpu.sync_copy(bufs.at[b], acc_sh.at[col_v.at[g]],
                                    add=True)

                    @pl.when(g + NB < hch)
                    def _():
                        pltpu.async_copy(
                            xs_hbm.at[row_v.at[g + NB]], bufs.at[b], sems[b])
                return 0
            lax.fori_loop(0, hch // NB, body, 0)
        plsc.subcore_barrier()

        pltpu.sync_copy(acc_sh.at[pl.ds(sub * cr, cr)],
                        out_hbm.at[core, pl.ds(sub * cr, cr)])
        if tail:
            @pl.when(sub == 0)
            def _():
                pltpu.sync_copy(acc_sh.at[pl.ds(cr * NS, tail)],
                                out_hbm.at[core, pl.ds(cr * NS, tail)])

    return scat_k


def _make_prep(n, d, bn, n_pad, pad):
    nb = n_pad // bn

    def prep_k(x_ref, hist_ref, wl_ref, fcw_ref, w1_ref, b1_ref, w2_ref,
               b2_ref, alpha_ref, fcb_ref, xs_ref, s_ref, wcs_ref, rv_ref,
               acc):
        i = pl.program_id(0)
        gri = i * bn + lax.broadcasted_iota(jnp.int32, (bn, 1), 0)
        deg = (jnp.sum(hist_ref[0], axis=1, keepdims=True)
               - jnp.where(gri < pad, 1.0, 0.0))
        s = jnp.where(deg > 0.0, lax.rsqrt(jnp.maximum(deg, 1e-30)), 0.0)
        valid = gri < n
        xb = jnp.where(valid, x_ref[...], 0.0)
        xs_ref[...] = jnp.where(valid, xb * s, 0.0)
        s_ref[...] = s

        psum = jnp.sum(xb, axis=0, keepdims=True)

        @pl.when(i == 0)
        def _():
            acc[0:1, :] = psum

        @pl.when(i > 0)
        def _():
            acc[0:1, :] = acc[0:1, :] + psum

        @pl.when(i == nb - 1)
        def _():
            mean = acc[0:1, :] * (1.0 / n)
            sig = jax.nn.sigmoid(alpha_ref[...])
            cdims = (((1,), (1,)), ((), ()))
            h = jnp.maximum(
                lax.dot_general(mean, w1_ref[...], cdims,
                                preferred_element_type=jnp.float32)
                + b1_ref[...], 0.0)
            g = lax.dot_general(h, w2_ref[...], cdims,
                                preferred_element_type=jnp.float32) + b2_ref[...]
            gf = lax.dot_general(g, fcw_ref[...], cdims,
                                 preferred_element_type=jnp.float32)
            rv_ref[...] = (1.0 - sig) * gf + fcb_ref[...]
            wcs_ref[...] = sig * lax.dot_general(
                wl_ref[...], fcw_ref[...], cdims,
                preferred_element_type=jnp.float32)

    full = lambda i: (0, 0)
    return pl.pallas_call(
        prep_k,
        grid=(nb,),
        in_specs=[
            pl.BlockSpec((bn, d), lambda i: (i, 0)),
            pl.BlockSpec((1, bn, 2), lambda i: (i, 0, 0)),
            pl.BlockSpec((d, d), full),
            pl.BlockSpec((d, d), full),
            pl.BlockSpec((d, d), full),
            pl.BlockSpec((1, d), full),
            pl.BlockSpec((d, d), full),
            pl.BlockSpec((1, d), full),
            pl.BlockSpec((1, 1), full),
            pl.BlockSpec((1, d), full),
        ],
        out_specs=[
            pl.BlockSpec((bn, d), lambda i: (i, 0)),
            pl.BlockSpec((bn, 1), lambda i: (i, 0)),
            pl.BlockSpec((d, d), full),
            pl.BlockSpec((1, d), full),
        ],
        out_shape=[
            jax.ShapeDtypeStruct((n_pad, d), jnp.float32),
            jax.ShapeDtypeStruct((n_pad, 1), jnp.float32),
            jax.ShapeDtypeStruct((d, d), jnp.float32),
            jax.ShapeDtypeStruct((1, d), jnp.float32),
        ],
        scratch_shapes=[pltpu.VMEM((8, d), jnp.float32)],
    )


def _make_final(n, d, bn):
    nb = n // bn

    def fin_k(acc_ref, s_ref, wcs_ref, rv_ref, out_ref):
        a = (acc_ref[0] + acc_ref[1]) * s_ref[...]
        out_ref[...] = jnp.dot(a, wcs_ref[...],
                               preferred_element_type=jnp.float32) + rv_ref[...]

    full = lambda i: (0, 0)
    return pl.pallas_call(
        fin_k,
        grid=(nb,),
        in_specs=[
            pl.BlockSpec((NC, bn, d), lambda i: (0, i, 0)),
            pl.BlockSpec((bn, 1), lambda i: (i, 0)),
            pl.BlockSpec((d, d), full),
            pl.BlockSpec((1, d), full),
        ],
        out_specs=pl.BlockSpec((bn, d), lambda i: (i, 0)),
        out_shape=jax.ShapeDtypeStruct((n, d), jnp.float32),
    )


def kernel(x, edge_index, weight_local, w1, b1, w2, b2, alpha, fc_w, fc_b):
    n, d = x.shape
    e = edge_index.shape[1]
    bn = 1000
    bp = 1008
    n_pad = -(-n // bp) * bp

    row = edge_index[0].astype(jnp.int32)
    col = edge_index[1].astype(jnp.int32)

    nch = -(-e // (NW * K))
    nch += (-nch) % 4
    e_pad = NW * K * nch
    pad = e_pad - e
    assert pad < n and n + 8 <= n_pad
    if pad:
        ar = jnp.arange(pad, dtype=jnp.int32)
        row_d = n + (ar % (n_pad - n))
        col_d = ar % n
        if e % NW == 0 and pad % NW == 0:
            row_p = jnp.concatenate(
                [row.reshape(NW, -1), row_d.reshape(NW, -1)], axis=1)
            col_p = jnp.concatenate(
                [col.reshape(NW, -1), col_d.reshape(NW, -1)], axis=1)
            row3 = row_p.reshape(NW, nch, K)
            col3 = col_p.reshape(NW, nch, K)
        else:
            row3 = jnp.concatenate([row, row_d]).reshape(NW, nch, K)
            col3 = jnp.concatenate([col, col_d]).reshape(NW, nch, K)
    else:
        row3 = row.reshape(NW, nch, K)
        col3 = col.reshape(NW, nch, K)

    hists = _make_hist(n, nch)(col3).reshape(NC, n)
    histt = jnp.concatenate(
        [jnp.transpose(hists),
         jnp.zeros((n_pad - n, NC), jnp.float32)]).reshape(-1, bp, NC)

    xs, s, wcs, rv = _make_prep(n, d, bp, n_pad, pad)(
        x, histt, weight_local, fc_w, w1, b1.reshape(1, d), w2,
        b2.reshape(1, d), alpha.reshape(1, 1), fc_b.reshape(1, d))

    accp = _make_scatter(n, nch, d)(xs, row3, col3)

    return _make_final(n, d, bn)(accp, s, wcs, rv)


# speedup vs baseline: 32.1832x; 1.0292x over previous
import functools

import jax
import jax.numpy as jnp
from jax import lax
from jax.experimental import pallas as pl
from jax.experimental.pallas import tpu as pltpu
from jax.experimental.pallas import tpu_sc as plsc

NC = 2
NS = 16
NW = NC * NS
K = 128
TRASH = 16


def _make_hist(n, nch):
    zr = (n // NS) & ~7
    tail = n - zr * NS
    mesh = plsc.VectorSubcoreMesh(core_axis_name="c", subcore_axis_name="s")

    @functools.partial(
        pl.kernel, mesh=mesh,
        out_type=jax.ShapeDtypeStruct((NC * n,), jnp.float32),
        scratch_types=[
            pltpu.VMEM_SHARED((n,), jnp.float32),
            pltpu.VMEM((nch, K), jnp.int32),
            pltpu.VMEM((K,), jnp.float32),
            pltpu.VMEM((zr,), jnp.float32),
            pltpu.SemaphoreType.DMA,
        ],
    )
    def hist_k(col_hbm, out_hbm, hist_sh, col_v, ones_v, z_v, hsem):
        core = lax.axis_index("c")
        sub = lax.axis_index("s")
        wid = sub * NC + core

        def fill_ones(i, _):
            ones_v[pl.ds(i * 16, 16)] = jnp.ones((16,), jnp.float32)
            return 0
        lax.fori_loop(0, K // 16, fill_ones, 0)

        def fill_zero(i, _):
            z_v[pl.ds(i * 16, 16)] = jnp.zeros((16,), jnp.float32)
            return 0
        lax.fori_loop(0, zr // 16, fill_zero, 0)

        pltpu.sync_copy(z_v, hist_sh.at[pl.ds(sub * zr, zr)])
        if tail:
            @pl.when(sub == 0)
            def _():
                pltpu.sync_copy(z_v.at[pl.ds(0, tail)],
                                hist_sh.at[pl.ds(zr * NS, tail)])
        pltpu.sync_copy(col_hbm.at[wid], col_v)
        plsc.subcore_barrier()

        def body(o, _):
            for q in range(4):
                pltpu.async_copy(ones_v, hist_sh.at[col_v.at[o * 4 + q]],
                                 hsem, add=True)
            for q in range(4):
                pltpu.make_async_copy(
                    ones_v, hist_sh.at[col_v.at[o * 4]], hsem).wait()
            return 0
        lax.fori_loop(0, nch // 4, body, 0)
        rem = nch % 4
        for q in range(rem):
            pltpu.sync_copy(ones_v, hist_sh.at[col_v.at[nch - rem + q]],
                            add=True)
        plsc.subcore_barrier()

        pltpu.sync_copy(hist_sh.at[pl.ds(sub * zr, zr)], z_v)
        pltpu.sync_copy(z_v, out_hbm.at[pl.ds(core * n + sub * zr, zr)])
        if tail:
            @pl.when(sub == 0)
            def _():
                pltpu.sync_copy(hist_sh.at[pl.ds(zr * NS, tail)],
                                z_v.at[pl.ds(0, tail)])
                pltpu.sync_copy(z_v.at[pl.ds(0, tail)],
                                out_hbm.at[pl.ds(core * n + zr * NS, tail)])

    return hist_k


def _make_scatter(n, nch, d):
    NB = 2
    assert nch % (2 * NB) == 0
    hch = nch // 2
    zr = 32
    cr = (n // NS) & ~7
    nz = cr // zr
    zrem = cr - nz * zr
    tail = n - cr * NS
    mesh = plsc.VectorSubcoreMesh(core_axis_name="c", subcore_axis_name="s")

    @functools.partial(
        pl.kernel, mesh=mesh,
        out_type=jax.ShapeDtypeStruct((NC, n, d), jnp.float32),
        scratch_types=[
            pltpu.VMEM_SHARED((n, d), jnp.float32),
            pltpu.VMEM((hch, K), jnp.int32),
            pltpu.VMEM((hch, K), jnp.int32),
            pltpu.VMEM((NB, K, d), jnp.float32),
            pltpu.VMEM((zr, d), jnp.float32),
            [pltpu.SemaphoreType.DMA] * NB,
        ],
    )
    def scat_k(xs_hbm, row_hbm, col_hbm, out_hbm,
               acc_sh, row_v, col_v, bufs, z_v, sems):
        core = lax.axis_index("c")
        sub = lax.axis_index("s")
        wid = sub * NC + core

        def zfill(r, _):
            for l in range(d // 16):
                z_v[r, pl.ds(l * 16, 16)] = jnp.zeros((16,), jnp.float32)
            return 0
        lax.fori_loop(0, zr, zfill, 0)

        def zcopy(j, _):
            pltpu.async_copy(z_v, acc_sh.at[pl.ds(sub * cr + j * zr, zr)],
                             sems[0])
            return 0
        lax.fori_loop(0, nz, zcopy, 0)
        if zrem:
            pltpu.sync_copy(z_v.at[pl.ds(0, zrem)],
                            acc_sh.at[pl.ds(sub * cr + nz * zr, zrem)])
        if tail:
            @pl.when(sub == 0)
            def _():
                pltpu.sync_copy(z_v.at[pl.ds(0, tail)],
                                acc_sh.at[pl.ds(cr * NS, tail)])
        pltpu.sync_copy(row_hbm.at[wid, pl.ds(0, hch)], row_v)
        pltpu.sync_copy(col_hbm.at[wid, pl.ds(0, hch)], col_v)

        def zdrain(j, _):
            pltpu.make_async_copy(
                z_v, acc_sh.at[pl.ds(sub * cr, zr)], sems[0]).wait()
            return 0
        lax.fori_loop(0, nz, zdrain, 0)
        plsc.subcore_barrier()

        for h in range(2):
            if h:
                pltpu.sync_copy(row_hbm.at[wid, pl.ds(h * hch, hch)], row_v)
                pltpu.sync_copy(col_hbm.at[wid, pl.ds(h * hch, hch)], col_v)

            for b in range(NB):
                pltpu.async_copy(xs_hbm.at[row_v.at[b]], bufs.at[b], sems[b])

            def body(o, _):
                for b in range(NB):
                    g = o * NB + b
                    pltpu.make_async_copy(
                        xs_hbm.at[row_v.at[g]], bufs.at[b], sems[b]).wait()
                    pltpu.sync_copy(bufs.at[b], acc_sh.at[col_v.at[g]],
                                    add=True)

                    @pl.when(g + NB < hch)
                    def _():
                        pltpu.async_copy(
                            xs_hbm.at[row_v.at[g + NB]], bufs.at[b], sems[b])
                return 0
            lax.fori_loop(0, hch // NB, body, 0)
        plsc.subcore_barrier()

        pltpu.sync_copy(acc_sh.at[pl.ds(sub * cr, cr)],
                        out_hbm.at[core, pl.ds(sub * cr, cr)])
        if tail:
            @pl.when(sub == 0)
            def _():
                pltpu.sync_copy(acc_sh.at[pl.ds(cr * NS, tail)],
                                out_hbm.at[core, pl.ds(cr * NS, tail)])

    return scat_k


def _make_prep(n, d, bn, n_pad, pad):
    nb = n_pad // bn

    def prep_k(x_ref, hist_ref, wl_ref, fcw_ref, w1_ref, b1_ref, w2_ref,
               b2_ref, alpha_ref, fcb_ref, xs_ref, s_ref, wcs_ref, rv_ref,
               acc):
        i = pl.program_id(0)
        gri = i * bn + lax.broadcasted_iota(jnp.int32, (bn, 1), 0)
        deg = (jnp.sum(hist_ref[0], axis=1, keepdims=True)
               - jnp.where(gri < pad, 1.0, 0.0))
        s = jnp.where(deg > 0.0, lax.rsqrt(jnp.maximum(deg, 1e-30)), 0.0)
        valid = gri < n
        xb = jnp.where(valid, x_ref[...], 0.0)
        xs_ref[...] = jnp.where(valid, xb * s, 0.0)
        s_ref[...] = s

        psum = jnp.sum(xb, axis=0, keepdims=True)

        @pl.when(i == 0)
        def _():
            acc[0:1, :] = psum

        @pl.when(i > 0)
        def _():
            acc[0:1, :] = acc[0:1, :] + psum

        @pl.when(i == nb - 1)
        def _():
            mean = acc[0:1, :] * (1.0 / n)
            sig = jax.nn.sigmoid(alpha_ref[...])
            cdims = (((1,), (1,)), ((), ()))
            h = jnp.maximum(
                lax.dot_general(mean, w1_ref[...], cdims,
                                preferred_element_type=jnp.float32)
                + b1_ref[...], 0.0)
            g = lax.dot_general(h, w2_ref[...], cdims,
                                preferred_element_type=jnp.float32) + b2_ref[...]
            gf = lax.dot_general(g, fcw_ref[...], cdims,
                                 preferred_element_type=jnp.float32)
            rv_ref[...] = (1.0 - sig) * gf + fcb_ref[...]
            wcs_ref[...] = sig * lax.dot_general(
                wl_ref[...], fcw_ref[...], cdims,
                preferred_element_type=jnp.float32)

    full = lambda i: (0, 0)
    return pl.pallas_call(
        prep_k,
        grid=(nb,),
        in_specs=[
            pl.BlockSpec((bn, d), lambda i: (i, 0)),
            pl.BlockSpec((1, bn, 2), lambda i: (i, 0, 0)),
            pl.BlockSpec((d, d), full),
            pl.BlockSpec((d, d), full),
            pl.BlockSpec((d, d), full),
            pl.BlockSpec((1, d), full),
            pl.BlockSpec((d, d), full),
            pl.BlockSpec((1, d), full),
            pl.BlockSpec((1, 1), full),
            pl.BlockSpec((1, d), full),
        ],
        out_specs=[
            pl.BlockSpec((bn, d), lambda i: (i, 0)),
            pl.BlockSpec((bn, 1), lambda i: (i, 0)),
            pl.BlockSpec((d, d), full),
            pl.BlockSpec((1, d), full),
        ],
        out_shape=[
            jax.ShapeDtypeStruct((n_pad, d), jnp.float32),
            jax.ShapeDtypeStruct((n_pad, 1), jnp.float32),
            jax.ShapeDtypeStruct((d, d), jnp.float32),
            jax.ShapeDtypeStruct((1, d), jnp.float32),
        ],
        scratch_shapes=[pltpu.VMEM((8, d), jnp.float32)],
    )


def _make_final(n, d, bn):
    nb = n // bn

    def fin_k(acc_ref, s_ref, wcs_ref, rv_ref, out_ref):
        a = (acc_ref[0] + acc_ref[1]) * s_ref[...]
        out_ref[...] = jnp.dot(a, wcs_ref[...],
                               preferred_element_type=jnp.float32) + rv_ref[...]

    full = lambda i: (0, 0)
    return pl.pallas_call(
        fin_k,
        grid=(nb,),
        in_specs=[
            pl.BlockSpec((NC, bn, d), lambda i: (0, i, 0)),
            pl.BlockSpec((bn, 1), lambda i: (i, 0)),
            pl.BlockSpec((d, d), full),
            pl.BlockSpec((1, d), full),
        ],
        out_specs=pl.BlockSpec((bn, d), lambda i: (i, 0)),
        out_shape=jax.ShapeDtypeStruct((n, d), jnp.float32),
    )


def kernel(x, edge_index, weight_local, w1, b1, w2, b2, alpha, fc_w, fc_b):
    n, d = x.shape
    e = edge_index.shape[1]
    bn = 2000
    bp = 2016
    n_pad = -(-n // bp) * bp

    row = edge_index[0].astype(jnp.int32)
    col = edge_index[1].astype(jnp.int32)

    nch = -(-e // (NW * K))
    nch += (-nch) % 4
    e_pad = NW * K * nch
    pad = e_pad - e
    assert pad < n and n + 8 <= n_pad
    if pad:
        ar = jnp.arange(pad, dtype=jnp.int32)
        row_d = n + (ar % (n_pad - n))
        col_d = ar % n
        if e % NW == 0 and pad % NW == 0:
            row_p = jnp.concatenate(
                [row.reshape(NW, -1), row_d.reshape(NW, -1)], axis=1)
            col_p = jnp.concatenate(
                [col.reshape(NW, -1), col_d.reshape(NW, -1)], axis=1)
            row3 = row_p.reshape(NW, nch, K)
            col3 = col_p.reshape(NW, nch, K)
        else:
            row3 = jnp.concatenate([row, row_d]).reshape(NW, nch, K)
            col3 = jnp.concatenate([col, col_d]).reshape(NW, nch, K)
    else:
        row3 = row.reshape(NW, nch, K)
        col3 = col.reshape(NW, nch, K)

    hists = _make_hist(n, nch)(col3).reshape(NC, n)
    histt = jnp.concatenate(
        [jnp.transpose(hists),
         jnp.zeros((n_pad - n, NC), jnp.float32)]).reshape(-1, bp, NC)

    xs, s, wcs, rv = _make_prep(n, d, bp, n_pad, pad)(
        x, histt, weight_local, fc_w, w1, b1.reshape(1, d), w2,
        b2.reshape(1, d), alpha.reshape(1, 1), fc_b.reshape(1, d))

    accp = _make_scatter(n, nch, d)(xs, row3, col3)

    return _make_final(n, d, bn)(accp, s, wcs, rv)
